# Initial kernel scaffold; baseline (speedup 1.0000x reference)
import jax
import jax.numpy as jnp
from jax.experimental import pallas as pl


def kernel(x, edge_index, W1l, b1l, W1r, W2l, b2l, W2r, W3l, b3l, W3r):
    raise NotImplementedError("write your pallas kernel here")


# speedup vs baseline: 5.2421x; 5.2421x over previous
import dataclasses
import functools

import jax
import jax.numpy as jnp
from jax import lax
from jax.experimental import pallas as pl
from jax.experimental.pallas import tpu as pltpu
from jax.experimental.pallas import tpu_sc as plsc

N = 10000
E = 320000
D = 128
L = 16

NC = 2
NS = 16
NW = NC * NS
EW = E // NW
CHUNK = 80
NCHUNK = EW // CHUNK
NP = 10240
RT = NP // NS

_mesh = plsc.VectorSubcoreMesh(core_axis_name="c", subcore_axis_name="s")


def _sc_body(with_counts, *refs):
    if with_counts:
        (x_hbm, src_hbm, dst_hbm, z_hbm, out_hbm, cnt_hbm, hist_hbm,
         src_v, dst_v, rows_v, hist_v, tmp_v,
         acc_sh) = refs
    else:
        (x_hbm, src_hbm, dst_hbm, z_hbm, out_hbm,
         src_v, dst_v, rows_v, acc_sh) = refs

    cid = lax.axis_index("c")
    sid = lax.axis_index("s")
    wid = cid * NS + sid
    row0 = sid * RT

    pltpu.sync_copy(z_hbm, rows_v)
    for k in range(RT // CHUNK):
        pltpu.sync_copy(rows_v, acc_sh.at[pl.ds(row0 + k * CHUNK, CHUNK)])
    if with_counts:
        @pl.loop(0, NP // L)
        def _(i):
            hist_v[pl.ds(i * L, L)] = jnp.zeros((L,), jnp.float32)

    plsc.subcore_barrier()

    base = wid * EW

    @pl.loop(0, NCHUNK)
    def _(j):
        off = base + j * CHUNK
        pltpu.sync_copy(src_hbm.at[pl.ds(off, CHUNK)], src_v)
        pltpu.sync_copy(dst_hbm.at[pl.ds(off, CHUNK)], dst_v)
        pltpu.sync_copy(x_hbm.at[src_v], rows_v)
        pltpu.sync_copy(rows_v, acc_sh.at[dst_v], add=True)
        if with_counts:
            for k in range(CHUNK // L):
                idx = dst_v[pl.ds(k * L, L)]
                cnts, last = plsc.scan_count(idx)
                plsc.addupdate_scatter(hist_v, [idx],
                                       cnts.astype(jnp.float32), mask=last)

    if with_counts:
        pltpu.sync_copy(hist_v, hist_hbm.at[pl.ds(wid * NP, NP)])
    plsc.subcore_barrier()

    for k in range(RT // CHUNK):
        pltpu.sync_copy(acc_sh.at[pl.ds(row0 + k * CHUNK, CHUNK)], rows_v)
        pltpu.sync_copy(rows_v, out_hbm.at[cid, pl.ds(row0 + k * CHUNK, CHUNK)])

    if with_counts:
        for j in range(NS):
            pltpu.sync_copy(hist_hbm.at[pl.ds((cid * NS + j) * NP + row0, RT)],
                            tmp_v.at[pl.ds(j * RT, RT)])
        lane_iota = jax.lax.iota(jnp.int32, L)
        zeros_i = jnp.zeros((L,), jnp.int32)

        for k in range(RT // CHUNK):
            @pl.loop(0, CHUNK // L)
            def _(b):
                tot = jnp.zeros((L,), jnp.float32)
                for j in range(NS):
                    tot += tmp_v[pl.ds(j * RT + k * CHUNK + b * L, L)]
                plsc.store_scatter(rows_v, [b * L + lane_iota, zeros_i], tot)

            pltpu.sync_copy(rows_v,
                            cnt_hbm.at[cid, pl.ds(row0 + k * CHUNK, CHUNK)])


def _make_sc_pass(with_counts):
    if with_counts:
        out_type = (jax.ShapeDtypeStruct((NC, NP, D), jnp.float32),
                    jax.ShapeDtypeStruct((NC, NP, D), jnp.float32),
                    jax.ShapeDtypeStruct((NW * NP,), jnp.float32))
    else:
        out_type = jax.ShapeDtypeStruct((NC, NP, D), jnp.float32)
    scratch = [
        pltpu.VMEM((CHUNK,), jnp.int32),
        pltpu.VMEM((CHUNK,), jnp.int32),
        pltpu.VMEM((CHUNK, D), jnp.float32),
    ]
    if with_counts:
        scratch += [
            pltpu.VMEM((NP,), jnp.float32),
            pltpu.VMEM((NS * RT,), jnp.float32),
        ]
    scratch.append(pltpu.VMEM_SHARED((NP, D), jnp.float32))
    cp = pltpu.CompilerParams()
    if "needs_layout_passes" in pltpu.CompilerParams.__dataclass_fields__:
        cp = dataclasses.replace(cp, needs_layout_passes=False)
    return pl.kernel(
        functools.partial(_sc_body, with_counts),
        out_type=out_type,
        mesh=_mesh,
        scratch_types=scratch,
        compiler_params=cp,
    )


_sc_pass_counts = _make_sc_pass(True)


R = 1000


def _t1_body(p_ref, c_ref, x_ref, wl_ref, bl_ref, wr_ref, o_ref):
    cnt = jnp.maximum(c_ref[0, :, 0:1] + c_ref[1, :, 0:1], 1.0)
    mean = (p_ref[0] + p_ref[1]) / cnt
    acc = lax.dot_general(mean, wl_ref[...], (((1,), (1,)), ((), ())),
                          preferred_element_type=jnp.float32)
    acc += lax.dot_general(x_ref[...], wr_ref[...], (((1,), (1,)), ((), ())),
                           preferred_element_type=jnp.float32)
    o_ref[...] = jnp.maximum(acc + bl_ref[...], 0.0)


def _t2_body(p_ref, c_ref, x_ref, w2l_ref, b2l_ref, w2r_ref,
             w3l_ref, b3l_ref, w3r_ref, h1_ref, h2_ref):
    cnt = jnp.maximum(c_ref[0, :, 0:1] + c_ref[1, :, 0:1], 1.0)
    mean = (p_ref[0] + p_ref[1]) / cnt
    xt = x_ref[...]
    a1 = lax.dot_general(mean, w2l_ref[...], (((1,), (1,)), ((), ())),
                         preferred_element_type=jnp.float32)
    a1 += lax.dot_general(xt, w2r_ref[...], (((1,), (1,)), ((), ())),
                          preferred_element_type=jnp.float32)
    h1_ref[...] = a1 + b2l_ref[...]
    a2 = lax.dot_general(mean, w3l_ref[...], (((1,), (1,)), ((), ())),
                         preferred_element_type=jnp.float32)
    a2 += lax.dot_general(xt, w3r_ref[...], (((1,), (1,)), ((), ())),
                          preferred_element_type=jnp.float32)
    h2_ref[...] = a2 + b3l_ref[...]


def _full(shape):
    return pl.BlockSpec(shape, lambda i: tuple(0 for _ in shape))


_p_spec = pl.BlockSpec((NC, R, D), lambda i: (0, i, 0))
_x_spec = pl.BlockSpec((R, D), lambda i: (i, 0))

_t1 = pl.pallas_call(
    _t1_body,
    grid=(N // R,),
    in_specs=[_p_spec, _p_spec, _x_spec, _full((D, D)), _full((1, D)),
              _full((D, D))],
    out_specs=_x_spec,
    out_shape=jax.ShapeDtypeStruct((N, D), jnp.float32),
)

_t2 = pl.pallas_call(
    _t2_body,
    grid=(N // R,),
    in_specs=[_p_spec, _p_spec, _x_spec, _full((D, D)), _full((1, D)),
              _full((D, D)), _full((D, D)), _full((1, D)), _full((D, D))],
    out_specs=[_x_spec, _x_spec],
    out_shape=[jax.ShapeDtypeStruct((N, D), jnp.float32),
               jax.ShapeDtypeStruct((N, D), jnp.float32)],
)


def kernel(x, edge_index, W1l, b1l, W1r, W2l, b2l, W2r, W3l, b3l, W3r):
    src = edge_index[0]
    dst = edge_index[1]
    z = jnp.zeros((CHUNK, D), jnp.float32)

    p1, cnts, _h1 = _sc_pass_counts(x, src, dst, z)
    xt = _t1(p1, cnts, x, W1l, b1l.reshape(1, D), W1r)
    p2, _c2, _h2 = _sc_pass_counts(xt, src, dst, z)
    h_, h = _t2(p2, cnts, xt, W2l, b2l.reshape(1, D), W2r,
                W3l, b3l.reshape(1, D), W3r)
    return (h_, h)
